# baseline (speedup 1.0000x reference)
import functools

import jax
import jax.numpy as jnp
from jax import lax
from jax.experimental import pallas as pl
from jax.experimental.pallas import tpu as pltpu
from jax.experimental.pallas import tpu_sc as plsc

EMBED_DIM = 64
ROWS_PER_CHUNK = 16


@functools.cache
def _make_gather(b_total: int, l: int, d: int):
    info = plsc.get_sparse_core_info()
    nc, ns = info.num_cores, info.num_subcores
    nw = nc * ns
    assert b_total % nw == 0
    rows_per_w = b_total // nw
    assert rows_per_w % ROWS_PER_CHUNK == 0
    n_steps = rows_per_w // ROWS_PER_CHUNK
    assert n_steps % 2 == 0

    mesh = plsc.VectorSubcoreMesh(core_axis_name="c", subcore_axis_name="s")

    @functools.partial(
        pl.kernel,
        mesh=mesh,
        out_type=jax.ShapeDtypeStruct((b_total, l, d), jnp.float32),
        scratch_types=[
            pltpu.VMEM((n_steps, ROWS_PER_CHUNK, l), jnp.int32),
            pltpu.VMEM((2, ROWS_PER_CHUNK, l, d), jnp.float32),
            pltpu.SemaphoreType.DMA,
            pltpu.SemaphoreType.DMA,
            pltpu.SemaphoreType.DMA,
        ],
        compiler_params=pltpu.CompilerParams(use_tc_tiling_on_sc=False),
    )
    def gather_kernel(idx_hbm, table_hbm, out_hbm, idx_v, rows_v, sg, ss0, ss1):
        wid = lax.axis_index("s") * nc + lax.axis_index("c")
        row_base = wid * rows_per_w
        ss = (ss0, ss1)

        pltpu.sync_copy(idx_hbm.at[wid], idx_v)

        def gather(step, slot):
            descs = [
                pltpu.async_copy(table_hbm.at[idx_v.at[step, j]],
                                 rows_v.at[slot, j], sg)
                for j in range(ROWS_PER_CHUNK)
            ]
            for desc in descs:
                desc.wait()

        def store_start(step, slot):
            dst = out_hbm.at[pl.ds(row_base + step * ROWS_PER_CHUNK,
                                   ROWS_PER_CHUNK)]
            pltpu.async_copy(rows_v.at[slot], dst, ss[slot])

        def store_wait(slot):
            dst = out_hbm.at[pl.ds(row_base, ROWS_PER_CHUNK)]
            pltpu.make_async_copy(rows_v.at[slot], dst, ss[slot]).wait()

        for b in (0, 1):
            gather(b, b)
            store_start(b, b)

        def body(g2, carry):
            for b in (0, 1):
                step = 2 * g2 + b
                store_wait(b)
                gather(step, b)
                store_start(step, b)
            return carry

        lax.fori_loop(1, n_steps // 2, body, 0)
        store_wait(0)
        store_wait(1)

    return gather_kernel


def kernel(channel_indices, table):
    b, l = channel_indices.shape
    info = plsc.get_sparse_core_info()
    nw = info.num_cores * info.num_subcores
    idx = channel_indices.reshape(nw, b // (nw * ROWS_PER_CHUNK),
                                  ROWS_PER_CHUNK, l)
    idx = idx.astype(jnp.int32)
    return _make_gather(b, l, table.shape[1])(idx, table)


# speedup vs baseline: 1.0014x; 1.0014x over previous
import functools

import jax
import jax.numpy as jnp
from jax import lax
from jax.experimental import pallas as pl
from jax.experimental.pallas import tpu as pltpu
from jax.experimental.pallas import tpu_sc as plsc

EMBED_DIM = 64
ROWS_PER_CHUNK = 16


@functools.cache
def _make_gather(b_total: int, l: int, d: int):
    info = plsc.get_sparse_core_info()
    nc, ns = info.num_cores, info.num_subcores
    nw = nc * ns
    assert b_total % nw == 0
    rows_per_w = b_total // nw
    assert rows_per_w % ROWS_PER_CHUNK == 0
    n_steps = rows_per_w // ROWS_PER_CHUNK
    assert n_steps % 2 == 0

    mesh = plsc.VectorSubcoreMesh(core_axis_name="c", subcore_axis_name="s")

    @functools.partial(
        pl.kernel,
        mesh=mesh,
        out_type=jax.ShapeDtypeStruct((b_total, l, d), jnp.float32),
        scratch_types=[
            pltpu.VMEM((rows_per_w, l), jnp.int32),
            pltpu.VMEM((2, ROWS_PER_CHUNK, l, d), jnp.float32),
            pltpu.SemaphoreType.DMA,
            pltpu.SemaphoreType.DMA,
            pltpu.SemaphoreType.DMA,
        ],
        compiler_params=pltpu.CompilerParams(use_tc_tiling_on_sc=False),
    )
    def gather_kernel(idx_hbm, table_hbm, out_hbm, idx_v, rows_v, sg, ss0, ss1):
        wid = lax.axis_index("s") * nc + lax.axis_index("c")
        row_base = wid * rows_per_w
        ss = (ss0, ss1)

        pltpu.sync_copy(idx_hbm.at[pl.ds(row_base, rows_per_w)], idx_v)

        def gather(step, slot):
            descs = [
                pltpu.async_copy(
                    table_hbm.at[idx_v.at[step * ROWS_PER_CHUNK + j]],
                    rows_v.at[slot, j], sg)
                for j in range(ROWS_PER_CHUNK)
            ]
            for desc in descs:
                desc.wait()

        def store_start(step, slot):
            dst = out_hbm.at[pl.ds(row_base + step * ROWS_PER_CHUNK,
                                   ROWS_PER_CHUNK)]
            pltpu.async_copy(rows_v.at[slot], dst, ss[slot])

        def store_wait(slot):
            dst = out_hbm.at[pl.ds(row_base, ROWS_PER_CHUNK)]
            pltpu.make_async_copy(rows_v.at[slot], dst, ss[slot]).wait()

        for b in (0, 1):
            gather(b, b)
            store_start(b, b)

        def body(g2, carry):
            for b in (0, 1):
                step = 2 * g2 + b
                store_wait(b)
                gather(step, b)
                store_start(step, b)
            return carry

        lax.fori_loop(1, n_steps // 2, body, 0)
        store_wait(0)
        store_wait(1)

    return gather_kernel


def kernel(channel_indices, table):
    b, l = channel_indices.shape
    info = plsc.get_sparse_core_info()
    nw = info.num_cores * info.num_subcores
    idx = channel_indices.astype(jnp.int32)
    return _make_gather(b, l, table.shape[1])(idx, table)


# speedup vs baseline: 1.7334x; 1.7310x over previous
import functools

import jax
import jax.numpy as jnp
from jax import lax
from jax.experimental import pallas as pl
from jax.experimental.pallas import tpu as pltpu
from jax.experimental.pallas import tpu_sc as plsc

EMBED_DIM = 64
ROWS_PER_CHUNK = 8


@functools.cache
def _make_gather(b_total: int, l: int, d: int):
    info = plsc.get_sparse_core_info()
    nc, ns = info.num_cores, info.num_subcores
    nw = nc * ns
    assert b_total % nw == 0
    rows_per_w = b_total // nw
    assert rows_per_w % ROWS_PER_CHUNK == 0
    n_steps = rows_per_w // ROWS_PER_CHUNK
    assert n_steps % 2 == 0

    mesh = plsc.VectorSubcoreMesh(core_axis_name="c", subcore_axis_name="s")

    @functools.partial(
        pl.kernel,
        mesh=mesh,
        out_type=jax.ShapeDtypeStruct((b_total, l, d), jnp.float32),
        scratch_types=[
            pltpu.VMEM((rows_per_w, l), jnp.int32),
            pltpu.VMEM_SHARED((144, d), jnp.float32),
            pltpu.VMEM((2, ROWS_PER_CHUNK, l, d), jnp.float32),
            pltpu.SemaphoreType.DMA,
            pltpu.SemaphoreType.DMA,
            pltpu.SemaphoreType.DMA,
        ],
        compiler_params=pltpu.CompilerParams(use_tc_tiling_on_sc=False),
    )
    def gather_kernel(idx_hbm, table_hbm, out_hbm, idx_v, table_v, rows_v,
                      sg, ss0, ss1):
        wid = lax.axis_index("s") * nc + lax.axis_index("c")
        row_base = wid * rows_per_w
        ss = (ss0, ss1)

        pltpu.sync_copy(idx_hbm.at[pl.ds(row_base, rows_per_w)], idx_v)
        pltpu.sync_copy(table_hbm, table_v)

        def gather(step, slot):
            descs = [
                pltpu.async_copy(
                    table_v.at[idx_v.at[step * ROWS_PER_CHUNK + j]],
                    rows_v.at[slot, j], sg)
                for j in range(ROWS_PER_CHUNK)
            ]
            for desc in descs:
                desc.wait()

        def store_start(step, slot):
            dst = out_hbm.at[pl.ds(row_base + step * ROWS_PER_CHUNK,
                                   ROWS_PER_CHUNK)]
            pltpu.async_copy(rows_v.at[slot], dst, ss[slot])

        def store_wait(slot):
            dst = out_hbm.at[pl.ds(row_base, ROWS_PER_CHUNK)]
            pltpu.make_async_copy(rows_v.at[slot], dst, ss[slot]).wait()

        for b in (0, 1):
            gather(b, b)
            store_start(b, b)

        def body(g2, carry):
            for b in (0, 1):
                step = 2 * g2 + b
                store_wait(b)
                gather(step, b)
                store_start(step, b)
            return carry

        lax.fori_loop(1, n_steps // 2, body, 0)
        store_wait(0)
        store_wait(1)

    return gather_kernel


def kernel(channel_indices, table):
    b, l = channel_indices.shape
    info = plsc.get_sparse_core_info()
    nw = info.num_cores * info.num_subcores
    idx = channel_indices.astype(jnp.int32)
    return _make_gather(b, l, table.shape[1])(idx, table)


# speedup vs baseline: 1.7348x; 1.0008x over previous
import functools

import jax
import jax.numpy as jnp
from jax import lax
from jax.experimental import pallas as pl
from jax.experimental.pallas import tpu as pltpu
from jax.experimental.pallas import tpu_sc as plsc

ROWS_PER_CHUNK = 8


@functools.cache
def _make_gather(b_total: int, l: int, v: int, d: int):
    info = plsc.get_sparse_core_info()
    nc, ns = info.num_cores, info.num_subcores
    nw = nc * ns
    assert b_total % nw == 0
    rows_per_w = b_total // nw
    assert rows_per_w % ROWS_PER_CHUNK == 0
    n_steps = rows_per_w // ROWS_PER_CHUNK
    assert n_steps % 2 == 0

    mesh = plsc.VectorSubcoreMesh(core_axis_name="c", subcore_axis_name="s")

    @functools.partial(
        pl.kernel,
        mesh=mesh,
        out_type=jax.ShapeDtypeStruct((b_total, l, d), jnp.float32),
        scratch_types=[
            pltpu.VMEM((rows_per_w, l), jnp.int32),
            pltpu.VMEM_SHARED((v, d), jnp.float32),
            pltpu.VMEM((2, ROWS_PER_CHUNK, l, d), jnp.float32),
            pltpu.SemaphoreType.DMA,
            pltpu.SemaphoreType.DMA,
            pltpu.SemaphoreType.DMA,
        ],
        compiler_params=pltpu.CompilerParams(use_tc_tiling_on_sc=False),
    )
    def gather_kernel(idx_hbm, table_hbm, out_hbm, idx_v, table_v, rows_v,
                      sg, ss0, ss1):
        wid = lax.axis_index("s") * nc + lax.axis_index("c")
        row_base = wid * rows_per_w
        ss = (ss0, ss1)

        pltpu.sync_copy(idx_hbm.at[pl.ds(row_base, rows_per_w)], idx_v)
        pltpu.sync_copy(table_hbm, table_v)

        def gather(step, slot):
            descs = [
                pltpu.async_copy(
                    table_v.at[idx_v.at[step * ROWS_PER_CHUNK + j]],
                    rows_v.at[slot, j], sg)
                for j in range(ROWS_PER_CHUNK)
            ]
            for desc in descs:
                desc.wait()

        def store_start(step, slot):
            dst = out_hbm.at[pl.ds(row_base + step * ROWS_PER_CHUNK,
                                   ROWS_PER_CHUNK)]
            pltpu.async_copy(rows_v.at[slot], dst, ss[slot])

        def store_wait(slot):
            dst = out_hbm.at[pl.ds(row_base, ROWS_PER_CHUNK)]
            pltpu.make_async_copy(rows_v.at[slot], dst, ss[slot]).wait()

        for b in (0, 1):
            gather(b, b)
            store_start(b, b)

        def body(g2, carry):
            for b in (0, 1):
                step = 2 * g2 + b
                store_wait(b)
                gather(step, b)
                store_start(step, b)
            return carry

        lax.fori_loop(1, n_steps // 2, body, 0)
        store_wait(0)
        store_wait(1)

    return gather_kernel


def kernel(channel_indices, table):
    b, l = channel_indices.shape
    idx = channel_indices.astype(jnp.int32)
    return _make_gather(b, l, table.shape[0], table.shape[1])(idx, table)
